# Initial kernel scaffold; baseline (speedup 1.0000x reference)
import functools

import jax
import jax.numpy as jnp
from jax import lax
from jax.experimental import pallas as pl
from jax.experimental.pallas import tpu as pltpu
from jax.experimental.pallas import tpu_sc as plsc

_N = 10000
_E = 320000
_G = 64
_H = 256

_TILES = 16
_CHUNK = 80
_EPT = _E // _TILES
_NCHUNK = _EPT // _CHUNK
_RPT = _N // _TILES


def _make_scatter_add(fh):
  mesh = plsc.VectorSubcoreMesh(core_axis_name="c", subcore_axis_name="s")

  @functools.partial(
      pl.kernel,
      out_type=jax.ShapeDtypeStruct((2, _N, fh), jnp.float32),
      mesh=mesh,
      scratch_types=[
          pltpu.VMEM((_CHUNK,), jnp.int32),
          pltpu.VMEM((_CHUNK,), jnp.int32),
          pltpu.VMEM((_CHUNK, fh), jnp.float32),
          pltpu.VMEM_SHARED((_N, fh), jnp.float32),
          pltpu.SemaphoreType.DMA,
      ],
  )
  def sc_kernel(x2, gidx, dst, zrows, out, idxg_v, idxd_v, rows_v, agg_sh,
                sem):
    c = lax.axis_index("c")
    s = lax.axis_index("s")
    r0 = s * _RPT
    pltpu.sync_copy(zrows.at[pl.ds(r0, _RPT)], agg_sh.at[pl.ds(r0, _RPT)])
    plsc.subcore_barrier()
    e0 = s * _EPT

    def chunk(j, carry):
      base = e0 + j * _CHUNK
      pltpu.sync_copy(gidx.at[pl.ds(c * _E + base, _CHUNK)], idxg_v)
      pltpu.sync_copy(dst.at[pl.ds(base, _CHUNK)], idxd_v)
      pltpu.async_copy(x2.at[idxg_v], rows_v, sem).wait()
      pltpu.sync_copy(rows_v, agg_sh.at[idxd_v], add=True)
      return carry

    lax.fori_loop(0, _NCHUNK, chunk, 0)
    plsc.subcore_barrier()
    pltpu.sync_copy(agg_sh.at[pl.ds(r0, _RPT)],
                    out.at[c].at[pl.ds(r0, _RPT)])

  return sc_kernel


_scatter64 = _make_scatter_add(64)
_scatter128 = _make_scatter_add(128)


def _bn_relu(h, g, b):
  mu = jnp.mean(h, axis=0, keepdims=True)
  var = jnp.mean(h * h, axis=0, keepdims=True) - mu * mu
  return jnp.maximum((h - mu) * lax.rsqrt(var + 1e-5) * g + b, 0.0)


def _dense_body(eps_ref, x_ref, agg_ref, wa_ref, ba_ref, ga_ref, bea_ref,
                wb_ref, bb_ref, go_ref, beo_ref, out_ref):
  agg = jnp.concatenate([agg_ref[0], agg_ref[1]], axis=1)
  m = x_ref[...] * (1.0 + eps_ref[0, 0]) + agg
  h = jnp.dot(m, wa_ref[...], preferred_element_type=jnp.float32) + ba_ref[...]
  z = _bn_relu(h, ga_ref[...], bea_ref[...])
  h2 = jnp.dot(z, wb_ref[...], preferred_element_type=jnp.float32) + bb_ref[...]
  out_ref[...] = _bn_relu(h2, go_ref[...], beo_ref[...])


def _dense_pool_body(eps_ref, x_ref, agg_ref, wa_ref, ba_ref, ga_ref, bea_ref,
                     wb_ref, bb_ref, go_ref, beo_ref, batch_ref, wlin_ref,
                     blin_ref, out_ref):
  agg = jnp.concatenate([agg_ref[0], agg_ref[1]], axis=1)
  m = x_ref[...] * (1.0 + eps_ref[0, 0]) + agg
  h = jnp.dot(m, wa_ref[...], preferred_element_type=jnp.float32) + ba_ref[...]
  z = _bn_relu(h, ga_ref[...], bea_ref[...])
  h2 = jnp.dot(z, wb_ref[...], preferred_element_type=jnp.float32) + bb_ref[...]
  x3 = _bn_relu(h2, go_ref[...], beo_ref[...])
  b = batch_ref[...]
  seg = lax.broadcasted_iota(jnp.int32, (_G, _N), 0)
  mask = (b == seg).astype(jnp.float32)
  pooled = jnp.dot(mask, x3, preferred_element_type=jnp.float32)
  out_ref[...] = (jnp.dot(pooled, wlin_ref[...],
                          preferred_element_type=jnp.float32) + blin_ref[...])


def _specs(n):
  return [pl.BlockSpec(memory_space=pltpu.SMEM)] + [pl.BlockSpec()] * n


def _dense_call(eps, x, agg, wa, ba, ga, bea, wb, bb, go, beo):
  return pl.pallas_call(
      _dense_body,
      out_shape=jax.ShapeDtypeStruct((_N, _H), jnp.float32),
      in_specs=_specs(10),
  )(jnp.reshape(eps, (1, 1)), x, agg, wa, ba, ga, bea, wb, bb, go, beo)


def _dense_pool_call(eps, x, agg, wa, ba, ga, bea, wb, bb, go, beo, batch,
                     wlin, blin):
  return pl.pallas_call(
      _dense_pool_body,
      out_shape=jax.ShapeDtypeStruct((_G, wlin.shape[1]), jnp.float32),
      in_specs=_specs(13),
  )(jnp.reshape(eps, (1, 1)), x, agg, wa, ba, ga, bea, wb, bb, go, beo,
    batch, wlin, blin)


def _row2(v):
  return jnp.reshape(v, (1, -1))


def kernel(x, edge_index, batch,
           eps1, W1a, b1a, g1a, be1a, W1b, b1b, g1o, be1o,
           eps2, W2a, b2a, g2a, be2a, W2b, b2b, g2o, be2o,
           eps3, W3a, b3a, g3a, be3a, W3b, b3b, g3o, be3o,
           Wlin, blin):
  x = x.astype(jnp.float32)
  src = edge_index[0].astype(jnp.int32)
  dst = edge_index[1].astype(jnp.int32)
  gidx = jnp.concatenate([2 * src, 2 * src + 1])
  z64 = jnp.zeros((_N, 64), jnp.float32)
  z128 = jnp.zeros((_N, 128), jnp.float32)
  batch2 = jnp.reshape(batch.astype(jnp.int32), (1, _N))

  agg1 = _scatter64(x.reshape(2 * _N, 64), gidx, dst, z64)
  x1 = _dense_call(eps1, x, agg1, W1a, _row2(b1a), _row2(g1a), _row2(be1a),
                   W1b, _row2(b1b), _row2(g1o), _row2(be1o))
  agg2 = _scatter128(x1.reshape(2 * _N, 128), gidx, dst, z128)
  x2 = _dense_call(eps2, x1, agg2, W2a, _row2(b2a), _row2(g2a), _row2(be2a),
                   W2b, _row2(b2b), _row2(g2o), _row2(be2o))
  agg3 = _scatter128(x2.reshape(2 * _N, 128), gidx, dst, z128)
  return _dense_pool_call(eps3, x2, agg3, W3a, _row2(b3a), _row2(g3a),
                          _row2(be3a), W3b, _row2(b3b), _row2(g3o),
                          _row2(be3o), batch2, Wlin, _row2(blin))


# speedup vs baseline: 3.4956x; 3.4956x over previous
import functools

import jax
import jax.numpy as jnp
from jax import lax
from jax.experimental import pallas as pl
from jax.experimental.pallas import tpu as pltpu
from jax.experimental.pallas import tpu_sc as plsc

_N = 10000
_E = 320000
_G = 64
_H = 256

_TILES = 16
_CHUNK = 80
_EPT = _E // _TILES
_NCHUNK = _EPT // _CHUNK
_NP = 10240
_RPT = _NP // _TILES


def _make_scatter_add(split):
  fh = 128
  ept = _EPT if split else _EPT // 2
  nchunk = ept // _CHUNK
  mesh = plsc.VectorSubcoreMesh(core_axis_name="c", subcore_axis_name="s")

  @functools.partial(
      pl.kernel,
      out_type=jax.ShapeDtypeStruct((2, _NP, fh), jnp.float32),
      mesh=mesh,
      scratch_types=[
          pltpu.VMEM((_CHUNK,), jnp.int32),
          pltpu.VMEM((_CHUNK,), jnp.int32),
          pltpu.VMEM((_CHUNK, fh), jnp.float32),
          pltpu.VMEM_SHARED((_NP, fh), jnp.float32),
          pltpu.SemaphoreType.DMA,
      ],
  )
  def sc_kernel(x2, gidx, dst, zrows, out, idxg_v, idxd_v, rows_v, agg_sh,
                sem):
    c = lax.axis_index("c")
    s = lax.axis_index("s")
    r0 = s * _RPT
    pltpu.sync_copy(zrows.at[pl.ds(r0, _RPT)], agg_sh.at[pl.ds(r0, _RPT)])
    plsc.subcore_barrier()
    if split:
      e0 = s * ept
      g0 = c * _E + e0
    else:
      e0 = (c * 16 + s) * ept
      g0 = e0

    def chunk(j, carry):
      base = j * _CHUNK
      pltpu.sync_copy(gidx.at[pl.ds(g0 + base, _CHUNK)], idxg_v)
      pltpu.sync_copy(dst.at[pl.ds(e0 + base, _CHUNK)], idxd_v)
      pltpu.async_copy(x2.at[idxg_v], rows_v, sem).wait()
      pltpu.sync_copy(rows_v, agg_sh.at[idxd_v], add=True)
      return carry

    lax.fori_loop(0, nchunk, chunk, 0)
    plsc.subcore_barrier()
    pltpu.sync_copy(agg_sh.at[pl.ds(r0, _RPT)],
                    out.at[c].at[pl.ds(r0, _RPT)])

  return sc_kernel


_scatter_sum = _make_scatter_add(False)
_scatter_split = _make_scatter_add(True)


def _bn_relu(h, g, b):
  mu = jnp.mean(h, axis=0, keepdims=True)
  var = jnp.mean(h * h, axis=0, keepdims=True) - mu * mu
  return jnp.maximum((h - mu) * lax.rsqrt(var + 1e-5) * g + b, 0.0)


def _combine(agg_ref, split):
  if split:
    return jnp.concatenate([agg_ref[0, :_N], agg_ref[1, :_N]], axis=1)
  return agg_ref[0, :_N] + agg_ref[1, :_N]


def _dense_body(split, eps_ref, x_ref, agg_ref, wa_ref, ba_ref, ga_ref,
                bea_ref, wb_ref, bb_ref, go_ref, beo_ref, out_ref):
  agg = _combine(agg_ref, split)
  m = x_ref[...] * (1.0 + eps_ref[0, 0]) + agg
  h = jnp.dot(m, wa_ref[...], preferred_element_type=jnp.float32) + ba_ref[...]
  z = _bn_relu(h, ga_ref[...], bea_ref[...])
  h2 = jnp.dot(z, wb_ref[...], preferred_element_type=jnp.float32) + bb_ref[...]
  out_ref[...] = _bn_relu(h2, go_ref[...], beo_ref[...])


def _dense_pool_body(eps_ref, x_ref, agg_ref, wa_ref, ba_ref, ga_ref, bea_ref,
                     wb_ref, bb_ref, go_ref, beo_ref, batch_ref, wlin_ref,
                     blin_ref, out_ref):
  agg = _combine(agg_ref, True)
  m = x_ref[...] * (1.0 + eps_ref[0, 0]) + agg
  h = jnp.dot(m, wa_ref[...], preferred_element_type=jnp.float32) + ba_ref[...]
  z = _bn_relu(h, ga_ref[...], bea_ref[...])
  h2 = jnp.dot(z, wb_ref[...], preferred_element_type=jnp.float32) + bb_ref[...]
  x3 = _bn_relu(h2, go_ref[...], beo_ref[...])
  b = batch_ref[...]
  seg = lax.broadcasted_iota(jnp.int32, (_G, _N), 0)
  mask = (b == seg).astype(jnp.float32)
  pooled = jnp.dot(mask, x3, preferred_element_type=jnp.float32)
  out_ref[...] = (jnp.dot(pooled, wlin_ref[...],
                          preferred_element_type=jnp.float32) + blin_ref[...])


def _specs(n):
  return [pl.BlockSpec(memory_space=pltpu.SMEM)] + [pl.BlockSpec()] * n


def _dense_call(split, eps, x, agg, wa, ba, ga, bea, wb, bb, go, beo):
  return pl.pallas_call(
      functools.partial(_dense_body, split),
      out_shape=jax.ShapeDtypeStruct((_N, _H), jnp.float32),
      in_specs=_specs(10),
  )(jnp.reshape(eps, (1, 1)), x, agg, wa, ba, ga, bea, wb, bb, go, beo)


def _dense_pool_call(eps, x, agg, wa, ba, ga, bea, wb, bb, go, beo, batch,
                     wlin, blin):
  return pl.pallas_call(
      _dense_pool_body,
      out_shape=jax.ShapeDtypeStruct((_G, wlin.shape[1]), jnp.float32),
      in_specs=_specs(13),
  )(jnp.reshape(eps, (1, 1)), x, agg, wa, ba, ga, bea, wb, bb, go, beo,
    batch, wlin, blin)


def _row2(v):
  return jnp.reshape(v, (1, -1))


def kernel(x, edge_index, batch,
           eps1, W1a, b1a, g1a, be1a, W1b, b1b, g1o, be1o,
           eps2, W2a, b2a, g2a, be2a, W2b, b2b, g2o, be2o,
           eps3, W3a, b3a, g3a, be3a, W3b, b3b, g3o, be3o,
           Wlin, blin):
  x = x.astype(jnp.float32)
  src = edge_index[0].astype(jnp.int32)
  dst = edge_index[1].astype(jnp.int32)
  gidx2 = jnp.concatenate([2 * src, 2 * src + 1])
  z128 = jnp.zeros((_NP, 128), jnp.float32)
  batch2 = jnp.reshape(batch.astype(jnp.int32), (1, _N))

  agg1 = _scatter_sum(x, src, dst, z128)
  x1 = _dense_call(False, eps1, x, agg1, W1a, _row2(b1a), _row2(g1a),
                   _row2(be1a), W1b, _row2(b1b), _row2(g1o), _row2(be1o))
  agg2 = _scatter_split(x1.reshape(2 * _N, 128), gidx2, dst, z128)
  x2 = _dense_call(True, eps2, x1, agg2, W2a, _row2(b2a), _row2(g2a),
                   _row2(be2a), W2b, _row2(b2b), _row2(g2o), _row2(be2o))
  agg3 = _scatter_split(x2.reshape(2 * _N, 128), gidx2, dst, z128)
  return _dense_pool_call(eps3, x2, agg3, W3a, _row2(b3a), _row2(g3a),
                          _row2(be3a), W3b, _row2(b3b), _row2(g3o),
                          _row2(be3o), batch2, Wlin, _row2(blin))
